# Initial kernel scaffold; baseline (speedup 1.0000x reference)
import jax
import jax.numpy as jnp
from jax.experimental import pallas as pl


def kernel(x, edge_index, batch, W1a, b1a, g1, be1, W1b, b1b, W2a, b2a, g2, be2, W2b, b2b, Wl1, bl1, Wl2, bl2):
    raise NotImplementedError("write your pallas kernel here")


# speedup vs baseline: 4.5874x; 4.5874x over previous
import functools

import jax
import jax.numpy as jnp
from jax import lax
from jax.experimental import pallas as pl
from jax.experimental.pallas import tpu as pltpu
from jax.experimental.pallas import tpu_sc as plsc

N = 10000
E = 320000
D = 128
G = 64
OUT = 10

NC = 2
NS = 16
NW = NC * NS
EPT = E // NW
EK = 80
NCHUNK = EPT // EK
NP = 10240
RPS = NP // NS
WB = 128


def _sc_aggregate(h, src, dst):
    mesh = plsc.VectorSubcoreMesh(core_axis_name="c", subcore_axis_name="s")

    @functools.partial(
        pl.kernel,
        out_type=jax.ShapeDtypeStruct((NC * NP, D), jnp.float32),
        mesh=mesh,
        scratch_types=[
            pltpu.VMEM((EK,), jnp.int32),
            pltpu.VMEM((EK,), jnp.int32),
            pltpu.VMEM((EK, D), jnp.float32),
            pltpu.VMEM((WB, D), jnp.float32),
            pltpu.VMEM_SHARED((NP, D), jnp.float32),
            pltpu.SemaphoreType.DMA,
        ],
    )
    def agg_kernel(h_hbm, src_hbm, dst_hbm, out_hbm, src_v, dst_v, rows_v,
                   wb_v, acc_sh, sem):
        cid = lax.axis_index("c")
        sid = lax.axis_index("s")
        wid = sid * NC + cid

        zero16 = jnp.zeros((16,), jnp.float32)

        def zero_row(i, carry):
            for j in range(D // 16):
                wb_v[i, pl.ds(j * 16, 16)] = zero16
            return carry

        lax.fori_loop(0, WB, zero_row, 0)
        rbase = sid * RPS
        for t in range(RPS // WB):
            pltpu.sync_copy(wb_v, acc_sh.at[pl.ds(rbase + t * WB, WB)])
        plsc.subcore_barrier()

        ebase = wid * EPT

        def chunk(j, carry):
            base = ebase + j * EK
            pltpu.sync_copy(src_hbm.at[pl.ds(base, EK)], src_v)
            pltpu.sync_copy(dst_hbm.at[pl.ds(base, EK)], dst_v)
            pltpu.async_copy(h_hbm.at[src_v], rows_v, sem).wait()
            pltpu.sync_copy(rows_v, acc_sh.at[dst_v], add=True)
            return carry

        lax.fori_loop(0, NCHUNK, chunk, 0)
        plsc.subcore_barrier()

        obase = cid * NP + rbase
        for t in range(RPS // WB):
            pltpu.sync_copy(acc_sh.at[pl.ds(rbase + t * WB, WB)], wb_v)
            pltpu.sync_copy(wb_v, out_hbm.at[pl.ds(obase + t * WB, WB)])

    return agg_kernel(h, src, dst)


def _mlp_kernel(h_ref, p_ref, wa_ref, ba_ref, g_ref, be_ref, wb_ref, bb_ref,
                out_ref):
    z = h_ref[...] + p_ref[0:N, :] + p_ref[NP:NP + N, :]
    t = jax.lax.dot_general(z, wa_ref[...], (((1,), (0,)), ((), ())),
                            precision=lax.Precision.HIGHEST,
                            preferred_element_type=jnp.float32)
    t = t + ba_ref[...]
    m = jnp.mean(t, axis=0, keepdims=True)
    v = jnp.mean((t - m) * (t - m), axis=0, keepdims=True)
    t = (t - m) * jax.lax.rsqrt(v + 1e-5) * g_ref[...] + be_ref[...]
    t = jnp.maximum(t, 0.0)
    t = jax.lax.dot_general(t, wb_ref[...], (((1,), (0,)), ((), ())),
                            precision=lax.Precision.HIGHEST,
                            preferred_element_type=jnp.float32)
    out_ref[...] = jnp.maximum(t + bb_ref[...], 0.0)


def _tc_mlp(h, partials, Wa, ba, g, be, Wb, bb):
    return pl.pallas_call(
        _mlp_kernel,
        out_shape=jax.ShapeDtypeStruct((N, D), jnp.float32),
    )(h, partials, Wa, ba.reshape(1, -1), g.reshape(1, -1),
      be.reshape(1, -1), Wb, bb.reshape(1, -1))


def _head_kernel(h1_ref, h2_ref, batch_ref, wl1_ref, bl1_ref, wl2_ref,
                 bl2_ref, out_ref):
    seg = lax.broadcasted_iota(jnp.int32, (N, G), 1)
    onehot = jnp.where(seg == batch_ref[...], 1.0, 0.0).astype(jnp.float32)
    p1 = jax.lax.dot_general(onehot, h1_ref[...], (((0,), (0,)), ((), ())),
                             precision=lax.Precision.HIGHEST,
                             preferred_element_type=jnp.float32)
    p2 = jax.lax.dot_general(onehot, h2_ref[...], (((0,), (0,)), ((), ())),
                             precision=lax.Precision.HIGHEST,
                             preferred_element_type=jnp.float32)
    hcat = jnp.concatenate([p1, p2], axis=1)
    a = jax.lax.dot_general(hcat, wl1_ref[...], (((1,), (0,)), ((), ())),
                            precision=lax.Precision.HIGHEST,
                            preferred_element_type=jnp.float32)
    a = jnp.maximum(a + bl1_ref[...], 0.0)
    o = jax.lax.dot_general(a, wl2_ref[...], (((1,), (0,)), ((), ())),
                            precision=lax.Precision.HIGHEST,
                            preferred_element_type=jnp.float32)
    o = o + bl2_ref[...]
    mx = jnp.max(o, axis=1, keepdims=True)
    s = o - mx
    lse = jnp.log(jnp.sum(jnp.exp(s), axis=1, keepdims=True))
    out_ref[...] = s - lse


def _tc_head(h1, h2, batch, Wl1, bl1, Wl2, bl2):
    return pl.pallas_call(
        _head_kernel,
        out_shape=jax.ShapeDtypeStruct((G, OUT), jnp.float32),
    )(h1, h2, batch.reshape(N, 1), Wl1, bl1.reshape(1, -1), Wl2,
      bl2.reshape(1, -1))


def kernel(x, edge_index, batch, W1a, b1a, g1, be1, W1b, b1b, W2a, b2a, g2,
           be2, W2b, b2b, Wl1, bl1, Wl2, bl2):
    src = edge_index[0]
    dst = edge_index[1]
    a1 = _sc_aggregate(x, src, dst)
    h1 = _tc_mlp(x, a1, W1a, b1a, g1, be1, W1b, b1b)
    a2 = _sc_aggregate(h1, src, dst)
    h2 = _tc_mlp(h1, a2, W2a, b2a, g2, be2, W2b, b2b)
    return _tc_head(h1, h2, batch, Wl1, bl1, Wl2, bl2)
